# Initial kernel scaffold; baseline (speedup 1.0000x reference)
import jax
import jax.numpy as jnp
from jax.experimental import pallas as pl


def kernel(x, edge_index, W1l, W1r, b1, W2l, W2r, b2, W3l, W3r, b3):
    raise NotImplementedError("write your pallas kernel here")


# speedup vs baseline: 6.5418x; 6.5418x over previous
import functools

import jax
import jax.numpy as jnp
from jax import lax
from jax.experimental import pallas as pl
from jax.experimental.pallas import tpu as pltpu
from jax.experimental.pallas import tpu_sc as plsc

N = 10000
E = 320000
D = 128

NC = 2
NS = 16
NW = NC * NS
EPT = E // NW
K = 128
NFULL = EPT // K
TAIL = EPT - NFULL * K
NP = 10240
RPT = NP // NS
ZR = 128


@functools.cache
def _make_agg(with_cnt):
  mesh = plsc.VectorSubcoreMesh(
      core_axis_name="c", subcore_axis_name="s", num_cores=NC, num_subcores=NS
  )
  if with_cnt:
    out_type = [
        jax.ShapeDtypeStruct((NC, NP, D), jnp.float32),
        jax.ShapeDtypeStruct((NC, NP), jnp.float32),
    ]
  else:
    out_type = jax.ShapeDtypeStruct((NC, NP, D), jnp.float32)
  scratch = [
      pltpu.VMEM((K,), jnp.int32),
      pltpu.VMEM((K,), jnp.int32),
      pltpu.VMEM((K, D), jnp.float32),
      pltpu.VMEM((TAIL,), jnp.int32),
      pltpu.VMEM((TAIL,), jnp.int32),
      pltpu.VMEM((TAIL, D), jnp.float32),
      pltpu.VMEM((ZR, D), jnp.float32),
      pltpu.VMEM_SHARED((NP, D), jnp.float32),
      pltpu.SemaphoreType.DMA,
  ]
  if with_cnt:
    scratch += [
        pltpu.VMEM((K,), jnp.float32),
        pltpu.VMEM((TAIL,), jnp.float32),
        pltpu.VMEM((RPT,), jnp.float32),
        pltpu.VMEM_SHARED((NP,), jnp.float32),
    ]

  def body(*refs):
    if with_cnt:
      (h_hbm, src_hbm, dst_hbm, agg_hbm, cnt_hbm,
       src_b, dst_b, rows_b, tsrc_b, tdst_b, trows_b, zb, acc, sem,
       ones_b, tones_b, cb, cacc) = refs
    else:
      (h_hbm, src_hbm, dst_hbm, agg_hbm,
       src_b, dst_b, rows_b, tsrc_b, tdst_b, trows_b, zb, acc, sem) = refs

    c = lax.axis_index("c")
    s = lax.axis_index("s")
    wid = c * NS + s

    zeros16 = jnp.zeros((16,), jnp.float32)
    ones16 = jnp.ones((16,), jnp.float32)

    def zfill(i, carry):
      def zcol(j, carry2):
        zb[i, pl.ds(j * 16, 16)] = zeros16
        return carry2
      return lax.fori_loop(0, D // 16, zcol, carry)
    lax.fori_loop(0, ZR, zfill, 0)

    if with_cnt:
      def ofill(i, carry):
        ones_b[pl.ds(i * 16, 16)] = ones16
        return carry
      lax.fori_loop(0, K // 16, ofill, 0)
      tones_b[pl.ds(0, 16)] = ones16
      def czfill(i, carry):
        cb[pl.ds(i * 16, 16)] = zeros16
        return carry
      lax.fori_loop(0, RPT // 16, czfill, 0)

    r0 = s * RPT
    def zacc(t, carry):
      pltpu.sync_copy(zb, acc.at[pl.ds(r0 + t * ZR, ZR)])
      return carry
    lax.fori_loop(0, RPT // ZR, zacc, 0)
    if with_cnt:
      pltpu.sync_copy(cb, cacc.at[pl.ds(r0, RPT)])

    plsc.subcore_barrier()

    e0 = wid * EPT
    def chunk(i, carry):
      off = e0 + i * K
      pltpu.sync_copy(src_hbm.at[pl.ds(off, K)], src_b)
      pltpu.sync_copy(dst_hbm.at[pl.ds(off, K)], dst_b)
      pltpu.async_copy(h_hbm.at[src_b], rows_b, sem).wait()
      pltpu.sync_copy(rows_b, acc.at[dst_b], add=True)
      if with_cnt:
        pltpu.sync_copy(ones_b, cacc.at[dst_b], add=True)
      return carry
    lax.fori_loop(0, NFULL, chunk, 0)

    toff = e0 + NFULL * K
    pltpu.sync_copy(src_hbm.at[pl.ds(toff, TAIL)], tsrc_b)
    pltpu.sync_copy(dst_hbm.at[pl.ds(toff, TAIL)], tdst_b)
    pltpu.async_copy(h_hbm.at[tsrc_b], trows_b, sem).wait()
    pltpu.sync_copy(trows_b, acc.at[tdst_b], add=True)
    if with_cnt:
      pltpu.sync_copy(tones_b, cacc.at[tdst_b], add=True)

    plsc.subcore_barrier()

    def wout(t, carry):
      r = r0 + t * ZR
      pltpu.sync_copy(acc.at[pl.ds(r, ZR)], zb)
      pltpu.sync_copy(zb, agg_hbm.at[c, pl.ds(r, ZR)])
      return carry
    lax.fori_loop(0, RPT // ZR, wout, 0)
    if with_cnt:
      pltpu.sync_copy(cacc.at[pl.ds(r0, RPT)], cb)
      pltpu.sync_copy(cb, cnt_hbm.at[c, pl.ds(r0, RPT)])

  return pl.kernel(
      body, out_type=out_type, mesh=mesh, scratch_types=scratch,
      compiler_params=pltpu.CompilerParams(use_tc_tiling_on_sc=False),
      name="sc_edge_agg_cnt" if with_cnt else "sc_edge_agg")


BN = 2000


def _tc_first_body(x_ref, wl_ref, wr_ref, b_ref, hl_ref, hr_ref):
  xb = x_ref[...]
  hl_ref[...] = jnp.dot(xb, wl_ref[...], preferred_element_type=jnp.float32)
  hr_ref[...] = (
      jnp.dot(xb, wr_ref[...], preferred_element_type=jnp.float32) + b_ref[...]
  )


def _tc_first(x, wl, wr, b):
  grid = (N // BN,)
  return pl.pallas_call(
      _tc_first_body,
      grid=grid,
      in_specs=[
          pl.BlockSpec((BN, D), lambda i: (i, 0)),
          pl.BlockSpec((D, D), lambda i: (0, 0)),
          pl.BlockSpec((D, D), lambda i: (0, 0)),
          pl.BlockSpec((1, D), lambda i: (0, 0)),
      ],
      out_specs=[
          pl.BlockSpec((BN, D), lambda i: (i, 0)),
          pl.BlockSpec((BN, D), lambda i: (i, 0)),
      ],
      out_shape=[
          jax.ShapeDtypeStruct((N, D), jnp.float32),
          jax.ShapeDtypeStruct((N, D), jnp.float32),
      ],
      name="tc_first",
  )(x, wl, wr, b)


def _mean(agg0, agg1, cnt0, cnt1):
  cnt = jnp.maximum(cnt0 + cnt1, 1.0)
  return (agg0 + agg1) / cnt


def _tc_mid_body(agg0_ref, agg1_ref, cnt0_ref, cnt1_ref, hr_ref, wl_ref,
                 wr_ref, b_ref, hl_ref, hro_ref):
  mean = _mean(agg0_ref[0], agg1_ref[0], cnt0_ref[0], cnt1_ref[0])
  h = jax.nn.relu(mean + hr_ref[...])
  hl_ref[...] = jnp.dot(h, wl_ref[...], preferred_element_type=jnp.float32)
  hro_ref[...] = (
      jnp.dot(h, wr_ref[...], preferred_element_type=jnp.float32) + b_ref[...]
  )


def _tc_mid(agg, cnt3, hr, wl, wr, b):
  grid = (N // BN,)
  return pl.pallas_call(
      _tc_mid_body,
      grid=grid,
      in_specs=[
          pl.BlockSpec((1, BN, D), lambda i: (0, i, 0)),
          pl.BlockSpec((1, BN, D), lambda i: (1, i, 0)),
          pl.BlockSpec((1, BN, 1), lambda i: (0, i, 0)),
          pl.BlockSpec((1, BN, 1), lambda i: (1, i, 0)),
          pl.BlockSpec((BN, D), lambda i: (i, 0)),
          pl.BlockSpec((D, D), lambda i: (0, 0)),
          pl.BlockSpec((D, D), lambda i: (0, 0)),
          pl.BlockSpec((1, D), lambda i: (0, 0)),
      ],
      out_specs=[
          pl.BlockSpec((BN, D), lambda i: (i, 0)),
          pl.BlockSpec((BN, D), lambda i: (i, 0)),
      ],
      out_shape=[
          jax.ShapeDtypeStruct((N, D), jnp.float32),
          jax.ShapeDtypeStruct((N, D), jnp.float32),
      ],
      name="tc_mid",
  )(agg, agg, cnt3, cnt3, hr, wl, wr, b)


def _tc_last_body(agg0_ref, agg1_ref, cnt0_ref, cnt1_ref, hr_ref, out_ref):
  mean = _mean(agg0_ref[0], agg1_ref[0], cnt0_ref[0], cnt1_ref[0])
  out_ref[...] = mean + hr_ref[...]


def _tc_last(agg, cnt3, hr):
  grid = (N // BN,)
  return pl.pallas_call(
      _tc_last_body,
      grid=grid,
      in_specs=[
          pl.BlockSpec((1, BN, D), lambda i: (0, i, 0)),
          pl.BlockSpec((1, BN, D), lambda i: (1, i, 0)),
          pl.BlockSpec((1, BN, 1), lambda i: (0, i, 0)),
          pl.BlockSpec((1, BN, 1), lambda i: (1, i, 0)),
          pl.BlockSpec((BN, D), lambda i: (i, 0)),
      ],
      out_specs=pl.BlockSpec((BN, D), lambda i: (i, 0)),
      out_shape=jax.ShapeDtypeStruct((N, D), jnp.float32),
      name="tc_last",
  )(agg, agg, cnt3, cnt3, hr)


@jax.jit
def kernel(x, edge_index, W1l, W1r, b1, W2l, W2r, b2, W3l, W3r, b3):
  src = edge_index[0].astype(jnp.int32)
  dst = edge_index[1].astype(jnp.int32)
  b1r = b1.reshape(1, D)
  b2r = b2.reshape(1, D)
  b3r = b3.reshape(1, D)

  hl, hr = _tc_first(x, W1l, W1r, b1r)
  agg, cntw = _make_agg(True)(hl, src, dst)
  cnt3 = cntw.reshape(NC, NP, 1)

  hl, hr = _tc_mid(agg, cnt3, hr, W2l, W2r, b2r)
  agg = _make_agg(False)(hl, src, dst)

  hl, hr = _tc_mid(agg, cnt3, hr, W3l, W3r, b3r)
  agg = _make_agg(False)(hl, src, dst)

  return _tc_last(agg, cnt3, hr)


# speedup vs baseline: 10.4767x; 1.6015x over previous
import functools

import jax
import jax.numpy as jnp
from jax import lax
from jax.experimental import pallas as pl
from jax.experimental.pallas import tpu as pltpu
from jax.experimental.pallas import tpu_sc as plsc

N = 10000
E = 320000
D = 128
DH = D // 2

NC = 2
NS = 16
NW = NC * NS
K = 128
NCHUNK = E // K
CPT = NCHUNK // NW
XTRA = NCHUNK - CPT * NW
MAXC1 = 80
NB = 6
A = 2
GMAX = (MAXC1 + NB - 1) // NB * NB // NB
NP = 10240
RPT = NP // NS
ZR = 128
PADROWS = 2504


@functools.cache
def _make_agg(with_cnt):
  mesh = plsc.VectorSubcoreMesh(
      core_axis_name="c", subcore_axis_name="s", num_cores=NC, num_subcores=NS
  )
  out_type = [
      jax.ShapeDtypeStruct((NC, NP, DH), jnp.float32),
      jax.ShapeDtypeStruct((NC, NP, DH), jnp.float32),
  ]
  if with_cnt:
    out_type.append(jax.ShapeDtypeStruct((NC, NP), jnp.float32))
  scratch = [
      pltpu.VMEM((MAXC1, K), jnp.int32),
      pltpu.VMEM((MAXC1, K), jnp.int32),
  ]
  scratch += [pltpu.VMEM((K, DH), jnp.float32) for _ in range(NB)]
  scratch += [pltpu.SemaphoreType.DMA for _ in range(NB)]
  scratch += [pltpu.SemaphoreType.DMA for _ in range(NB)]
  scratch += [pltpu.VMEM_SHARED((NP, DH), jnp.float32)]
  if with_cnt:
    scratch += [
        pltpu.VMEM((K,), jnp.float32),
        pltpu.VMEM((RPT,), jnp.float32),
        pltpu.VMEM_SHARED((NP,), jnp.float32),
        pltpu.SemaphoreType.DMA,
    ]

  def body(*refs):
    if with_cnt:
      (hA_hbm, hB_hbm, src_hbm, dst_hbm, aggA_hbm, aggB_hbm, cnt_hbm,
       srcv, dstv, *rest) = refs
      ones_b, cb, cacc, csem = rest[3 * NB + 1:]
    else:
      (hA_hbm, hB_hbm, src_hbm, dst_hbm, aggA_hbm, aggB_hbm,
       srcv, dstv, *rest) = refs
    rows = rest[:NB]
    gsem = rest[NB:2 * NB]
    ssem = rest[2 * NB:3 * NB]
    acc = rest[3 * NB]

    c = lax.axis_index("c")
    s = lax.axis_index("s")
    wid = c * NS + s
    nch = CPT + (wid < XTRA).astype(jnp.int32)
    cb0 = wid * CPT + jnp.minimum(wid, XTRA)
    r0 = s * RPT

    zeros16 = jnp.zeros((16,), jnp.float32)
    ones16 = jnp.ones((16,), jnp.float32)

    def pload(t, carry):
      pltpu.sync_copy(src_hbm.at[pl.ds(cb0 + t * 16, 16)],
                      srcv.at[pl.ds(t * 16, 16)])
      pltpu.sync_copy(dst_hbm.at[pl.ds(cb0 + t * 16, 16)],
                      dstv.at[pl.ds(t * 16, 16)])
      return carry
    lax.fori_loop(0, MAXC1 // 16, pload, 0)

    if with_cnt:
      def ofill(i, carry):
        ones_b[pl.ds(i * 16, 16)] = ones16
        return carry
      lax.fori_loop(0, K // 16, ofill, 0)
      def czfill(i, carry):
        cb[pl.ds(i * 16, 16)] = zeros16
        return carry
      lax.fori_loop(0, RPT // 16, czfill, 0)

    def run_phase(h_hbm, agg_hbm, do_cnt):
      def zfill(i, carry):
        def zcol(j, carry2):
          rows[0][i, pl.ds(j * 16, 16)] = zeros16
          return carry2
        return lax.fori_loop(0, DH // 16, zcol, carry)
      lax.fori_loop(0, ZR, zfill, 0)

      for t in range(RPT // ZR):
        pltpu.sync_copy(rows[0], acc.at[pl.ds(r0 + t * ZR, ZR)])
      if do_cnt:
        pltpu.sync_copy(cb, cacc.at[pl.ds(r0, RPT)])

      plsc.subcore_barrier()

      for j in range(A):
        pltpu.async_copy(h_hbm.at[srcv.at[j]], rows[j], gsem[j])

      def group(g, carry):
        for b in range(NB):
          i = g * NB + b
          j = i + A
          sj = (b + A) % NB

          @pl.when(j < nch)
          def _():
            @pl.when(j >= NB)
            def _():
              pltpu.make_async_copy(
                  rows[sj], acc.at[dstv.at[0]], ssem[sj]).wait()
            pltpu.async_copy(h_hbm.at[srcv.at[j]], rows[sj], gsem[sj])

          @pl.when(i < nch)
          def _():
            pltpu.make_async_copy(
                h_hbm.at[srcv.at[0]], rows[b], gsem[b]).wait()
            pltpu.async_copy(rows[b], acc.at[dstv.at[i]], ssem[b], add=True)
            if do_cnt:
              pltpu.async_copy(ones_b, cacc.at[dstv.at[i]], csem, add=True)
        return carry

      lax.fori_loop(0, GMAX, group, 0)

      for b in range(NB):
        pltpu.make_async_copy(rows[b], acc.at[dstv.at[0]], ssem[b]).wait()
      if do_cnt:
        def cdrain(i, carry):
          pltpu.make_async_copy(ones_b, cacc.at[dstv.at[0]], csem).wait()
          return carry
        lax.fori_loop(0, nch, cdrain, 0)

      plsc.subcore_barrier()

      def wout(t, carry):
        pltpu.sync_copy(acc.at[pl.ds(r0 + t * ZR, ZR)], rows[0])
        pltpu.sync_copy(rows[0], agg_hbm.at[c, pl.ds(r0 + t * ZR, ZR)])
        return carry
      lax.fori_loop(0, RPT // ZR, wout, 0)
      if do_cnt:
        pltpu.sync_copy(cacc.at[pl.ds(r0, RPT)], cb)
        pltpu.sync_copy(cb, cnt_hbm.at[c, pl.ds(r0, RPT)])

      plsc.subcore_barrier()

    run_phase(hA_hbm, aggA_hbm, with_cnt)
    run_phase(hB_hbm, aggB_hbm, False)

  return pl.kernel(
      body, out_type=out_type, mesh=mesh, scratch_types=scratch,
      compiler_params=pltpu.CompilerParams(use_tc_tiling_on_sc=False),
      name="sc_edge_agg_cnt" if with_cnt else "sc_edge_agg")


BN = 2000


def _tc_first_body(x_ref, wl_ref, wr_ref, b_ref, hla_ref, hlb_ref, hr_ref):
  xb = x_ref[...]
  hl = jnp.dot(xb, wl_ref[...], preferred_element_type=jnp.float32)
  hla_ref[...] = hl[:, :DH]
  hlb_ref[...] = hl[:, DH:]
  hr_ref[...] = (
      jnp.dot(xb, wr_ref[...], preferred_element_type=jnp.float32) + b_ref[...]
  )


def _tc_first(x, wl, wr, b):
  grid = (N // BN,)
  return pl.pallas_call(
      _tc_first_body,
      grid=grid,
      in_specs=[
          pl.BlockSpec((BN, D), lambda i: (i, 0)),
          pl.BlockSpec((D, D), lambda i: (0, 0)),
          pl.BlockSpec((D, D), lambda i: (0, 0)),
          pl.BlockSpec((1, D), lambda i: (0, 0)),
      ],
      out_specs=[
          pl.BlockSpec((BN, DH), lambda i: (i, 0)),
          pl.BlockSpec((BN, DH), lambda i: (i, 0)),
          pl.BlockSpec((BN, D), lambda i: (i, 0)),
      ],
      out_shape=[
          jax.ShapeDtypeStruct((N, DH), jnp.float32),
          jax.ShapeDtypeStruct((N, DH), jnp.float32),
          jax.ShapeDtypeStruct((N, D), jnp.float32),
      ],
      name="tc_first",
  )(x, wl, wr, b)


def _mean2(aggA0, aggA1, aggB0, aggB1, cnt0, cnt1):
  cnt = jnp.maximum(cnt0 + cnt1, 1.0)
  meanA = (aggA0 + aggA1) / cnt
  meanB = (aggB0 + aggB1) / cnt
  return jnp.concatenate([meanA, meanB], axis=1)


def _tc_mid_body(aggA0_ref, aggA1_ref, aggB0_ref, aggB1_ref, cnt0_ref,
                 cnt1_ref, hr_ref, wl_ref, wr_ref, b_ref,
                 hla_ref, hlb_ref, hro_ref):
  mean = _mean2(aggA0_ref[0], aggA1_ref[0], aggB0_ref[0], aggB1_ref[0],
                cnt0_ref[0], cnt1_ref[0])
  h = jax.nn.relu(mean + hr_ref[...])
  hl = jnp.dot(h, wl_ref[...], preferred_element_type=jnp.float32)
  hla_ref[...] = hl[:, :DH]
  hlb_ref[...] = hl[:, DH:]
  hro_ref[...] = (
      jnp.dot(h, wr_ref[...], preferred_element_type=jnp.float32) + b_ref[...]
  )


def _tc_mid(aggA, aggB, cnt3, hr, wl, wr, b):
  grid = (N // BN,)
  return pl.pallas_call(
      _tc_mid_body,
      grid=grid,
      in_specs=[
          pl.BlockSpec((1, BN, DH), lambda i: (0, i, 0)),
          pl.BlockSpec((1, BN, DH), lambda i: (1, i, 0)),
          pl.BlockSpec((1, BN, DH), lambda i: (0, i, 0)),
          pl.BlockSpec((1, BN, DH), lambda i: (1, i, 0)),
          pl.BlockSpec((1, BN, 1), lambda i: (0, i, 0)),
          pl.BlockSpec((1, BN, 1), lambda i: (1, i, 0)),
          pl.BlockSpec((BN, D), lambda i: (i, 0)),
          pl.BlockSpec((D, D), lambda i: (0, 0)),
          pl.BlockSpec((D, D), lambda i: (0, 0)),
          pl.BlockSpec((1, D), lambda i: (0, 0)),
      ],
      out_specs=[
          pl.BlockSpec((BN, DH), lambda i: (i, 0)),
          pl.BlockSpec((BN, DH), lambda i: (i, 0)),
          pl.BlockSpec((BN, D), lambda i: (i, 0)),
      ],
      out_shape=[
          jax.ShapeDtypeStruct((N, DH), jnp.float32),
          jax.ShapeDtypeStruct((N, DH), jnp.float32),
          jax.ShapeDtypeStruct((N, D), jnp.float32),
      ],
      name="tc_mid",
  )(aggA, aggA, aggB, aggB, cnt3, cnt3, hr, wl, wr, b)


def _tc_last_body(aggA0_ref, aggA1_ref, aggB0_ref, aggB1_ref, cnt0_ref,
                  cnt1_ref, hr_ref, out_ref):
  mean = _mean2(aggA0_ref[0], aggA1_ref[0], aggB0_ref[0], aggB1_ref[0],
                cnt0_ref[0], cnt1_ref[0])
  out_ref[...] = mean + hr_ref[...]


def _tc_last(aggA, aggB, cnt3, hr):
  grid = (N // BN,)
  return pl.pallas_call(
      _tc_last_body,
      grid=grid,
      in_specs=[
          pl.BlockSpec((1, BN, DH), lambda i: (0, i, 0)),
          pl.BlockSpec((1, BN, DH), lambda i: (1, i, 0)),
          pl.BlockSpec((1, BN, DH), lambda i: (0, i, 0)),
          pl.BlockSpec((1, BN, DH), lambda i: (1, i, 0)),
          pl.BlockSpec((1, BN, 1), lambda i: (0, i, 0)),
          pl.BlockSpec((1, BN, 1), lambda i: (1, i, 0)),
          pl.BlockSpec((BN, D), lambda i: (i, 0)),
      ],
      out_specs=pl.BlockSpec((BN, D), lambda i: (i, 0)),
      out_shape=jax.ShapeDtypeStruct((N, D), jnp.float32),
      name="tc_last",
  )(aggA, aggA, aggB, aggB, cnt3, cnt3, hr)


@jax.jit
def kernel(x, edge_index, W1l, W1r, b1, W2l, W2r, b2, W3l, W3r, b3):
  pad = jnp.zeros(((PADROWS - NCHUNK) * K,), jnp.int32)
  src = jnp.concatenate([edge_index[0].astype(jnp.int32), pad]).reshape(
      PADROWS, K)
  dst = jnp.concatenate([edge_index[1].astype(jnp.int32), pad]).reshape(
      PADROWS, K)
  b1r = b1.reshape(1, D)
  b2r = b2.reshape(1, D)
  b3r = b3.reshape(1, D)

  hla, hlb, hr = _tc_first(x, W1l, W1r, b1r)
  aggA, aggB, cntw = _make_agg(True)(hla, hlb, src, dst)
  cnt3 = cntw.reshape(NC, NP, 1)

  hla, hlb, hr = _tc_mid(aggA, aggB, cnt3, hr, W2l, W2r, b2r)
  aggA, aggB = _make_agg(False)(hla, hlb, src, dst)

  hla, hlb, hr = _tc_mid(aggA, aggB, cnt3, hr, W3l, W3r, b3r)
  aggA, aggB = _make_agg(False)(hla, hlb, src, dst)

  return _tc_last(aggA, aggB, cnt3, hr)


# speedup vs baseline: 11.6029x; 1.1075x over previous
import functools

import jax
import jax.numpy as jnp
from jax import lax
from jax.experimental import pallas as pl
from jax.experimental.pallas import tpu as pltpu
from jax.experimental.pallas import tpu_sc as plsc

N = 10000
E = 320000
D = 128
DH = D // 2

NC = 2
NS = 16
NW = NC * NS
K = 128
NCHUNK = E // K
CPT = NCHUNK // NW
XTRA = NCHUNK - CPT * NW
MAXC1 = 80
NB = 6
A = 3
GMAX = (MAXC1 + NB - 1) // NB * NB // NB
NP = 10240
RPT = NP // NS
ZR = 128
PADROWS = 2504


@functools.cache
def _make_agg(with_cnt):
  mesh = plsc.VectorSubcoreMesh(
      core_axis_name="c", subcore_axis_name="s", num_cores=NC, num_subcores=NS
  )
  out_type = [
      jax.ShapeDtypeStruct((NC, NP, DH), jnp.float32),
      jax.ShapeDtypeStruct((NC, NP, DH), jnp.float32),
  ]
  if with_cnt:
    out_type.append(jax.ShapeDtypeStruct((NC, NP), jnp.float32))
  scratch = [
      pltpu.VMEM((MAXC1, K), jnp.int32),
      pltpu.VMEM((MAXC1, K), jnp.int32),
  ]
  scratch += [pltpu.VMEM((K, DH), jnp.float32) for _ in range(NB)]
  scratch += [pltpu.SemaphoreType.DMA for _ in range(NB)]
  scratch += [pltpu.SemaphoreType.DMA for _ in range(NB)]
  scratch += [pltpu.VMEM((ZR, DH), jnp.float32)]
  scratch += [pltpu.VMEM_SHARED((NP, DH), jnp.float32)]
  if with_cnt:
    scratch += [
        pltpu.VMEM((K,), jnp.float32),
        pltpu.VMEM((RPT,), jnp.float32),
        pltpu.VMEM_SHARED((NP,), jnp.float32),
        pltpu.SemaphoreType.DMA,
    ]

  def body(*refs):
    if with_cnt:
      (hA_hbm, hB_hbm, src_hbm, dst_hbm, aggA_hbm, aggB_hbm, cnt_hbm,
       srcv, dstv, *rest) = refs
      ones_b, cb, cacc, csem = rest[3 * NB + 2:]
    else:
      (hA_hbm, hB_hbm, src_hbm, dst_hbm, aggA_hbm, aggB_hbm,
       srcv, dstv, *rest) = refs
    rows = rest[:NB]
    gsem = rest[NB:2 * NB]
    ssem = rest[2 * NB:3 * NB]
    zb = rest[3 * NB]
    acc = rest[3 * NB + 1]

    c = lax.axis_index("c")
    s = lax.axis_index("s")
    wid = c * NS + s
    nch = CPT + (wid < XTRA).astype(jnp.int32)
    cb0 = wid * CPT + jnp.minimum(wid, XTRA)
    r0 = s * RPT

    zeros16 = jnp.zeros((16,), jnp.float32)
    ones16 = jnp.ones((16,), jnp.float32)

    def pload(t, carry):
      pltpu.sync_copy(src_hbm.at[pl.ds(cb0 + t * 16, 16)],
                      srcv.at[pl.ds(t * 16, 16)])
      pltpu.sync_copy(dst_hbm.at[pl.ds(cb0 + t * 16, 16)],
                      dstv.at[pl.ds(t * 16, 16)])
      return carry
    lax.fori_loop(0, MAXC1 // 16, pload, 0)

    if with_cnt:
      def ofill(i, carry):
        ones_b[pl.ds(i * 16, 16)] = ones16
        return carry
      lax.fori_loop(0, K // 16, ofill, 0)
      def czfill(i, carry):
        cb[pl.ds(i * 16, 16)] = zeros16
        return carry
      lax.fori_loop(0, RPT // 16, czfill, 0)

    def zfill(i, carry):
      def zcol(j, carry2):
        zb[i, pl.ds(j * 16, 16)] = zeros16
        return carry2
      return lax.fori_loop(0, DH // 16, zcol, carry)
    lax.fori_loop(0, ZR, zfill, 0)

    def run_phase(h_hbm, agg_hbm, do_cnt):
      for t in range(RPT // ZR):
        pltpu.async_copy(zb, acc.at[pl.ds(r0 + t * ZR, ZR)], gsem[t])
      if do_cnt:
        pltpu.sync_copy(cb, cacc.at[pl.ds(r0, RPT)])
      for t in range(RPT // ZR):
        pltpu.make_async_copy(zb, acc.at[pl.ds(r0 + t * ZR, ZR)],
                              gsem[t]).wait()

      plsc.subcore_barrier()

      for j in range(A):
        pltpu.async_copy(h_hbm.at[srcv.at[j]], rows[j], gsem[j])

      def group(g, carry):
        for b in range(NB):
          i = g * NB + b
          j = i + A
          sj = (b + A) % NB

          @pl.when(j < nch)
          def _():
            @pl.when(j >= NB)
            def _():
              pltpu.make_async_copy(
                  rows[sj], acc.at[dstv.at[0]], ssem[sj]).wait()
            pltpu.async_copy(h_hbm.at[srcv.at[j]], rows[sj], gsem[sj])

          @pl.when(i < nch)
          def _():
            pltpu.make_async_copy(
                h_hbm.at[srcv.at[0]], rows[b], gsem[b]).wait()
            pltpu.async_copy(rows[b], acc.at[dstv.at[i]], ssem[b], add=True)
            if do_cnt:
              pltpu.async_copy(ones_b, cacc.at[dstv.at[i]], csem, add=True)
        return carry

      lax.fori_loop(0, GMAX, group, 0)

      for b in range(NB):
        pltpu.make_async_copy(rows[b], acc.at[dstv.at[0]], ssem[b]).wait()
      if do_cnt:
        def cdrain(i, carry):
          pltpu.make_async_copy(ones_b, cacc.at[dstv.at[0]], csem).wait()
          return carry
        lax.fori_loop(0, nch, cdrain, 0)

      plsc.subcore_barrier()

      for t in range(RPT // ZR):
        pltpu.sync_copy(acc.at[pl.ds(r0 + t * ZR, ZR)], rows[t])
        pltpu.async_copy(
            rows[t], agg_hbm.at[c, pl.ds(r0 + t * ZR, ZR)], gsem[t])
      if do_cnt:
        pltpu.sync_copy(cacc.at[pl.ds(r0, RPT)], cb)
        pltpu.sync_copy(cb, cnt_hbm.at[c, pl.ds(r0, RPT)])
      for t in range(RPT // ZR):
        pltpu.make_async_copy(
            rows[t], agg_hbm.at[c, pl.ds(r0 + t * ZR, ZR)], gsem[t]).wait()

      plsc.subcore_barrier()

    run_phase(hA_hbm, aggA_hbm, with_cnt)
    run_phase(hB_hbm, aggB_hbm, False)

  return pl.kernel(
      body, out_type=out_type, mesh=mesh, scratch_types=scratch,
      compiler_params=pltpu.CompilerParams(use_tc_tiling_on_sc=False),
      name="sc_edge_agg_cnt" if with_cnt else "sc_edge_agg")


BN = 2000


def _tc_first_body(x_ref, wl_ref, wr_ref, b_ref, hla_ref, hlb_ref, hr_ref):
  xb = x_ref[...]
  hl = jnp.dot(xb, wl_ref[...], preferred_element_type=jnp.float32)
  hla_ref[...] = hl[:, :DH]
  hlb_ref[...] = hl[:, DH:]
  hr_ref[...] = (
      jnp.dot(xb, wr_ref[...], preferred_element_type=jnp.float32) + b_ref[...]
  )


def _tc_first(x, wl, wr, b):
  grid = (N // BN,)
  return pl.pallas_call(
      _tc_first_body,
      grid=grid,
      in_specs=[
          pl.BlockSpec((BN, D), lambda i: (i, 0)),
          pl.BlockSpec((D, D), lambda i: (0, 0)),
          pl.BlockSpec((D, D), lambda i: (0, 0)),
          pl.BlockSpec((1, D), lambda i: (0, 0)),
      ],
      out_specs=[
          pl.BlockSpec((BN, DH), lambda i: (i, 0)),
          pl.BlockSpec((BN, DH), lambda i: (i, 0)),
          pl.BlockSpec((BN, D), lambda i: (i, 0)),
      ],
      out_shape=[
          jax.ShapeDtypeStruct((N, DH), jnp.float32),
          jax.ShapeDtypeStruct((N, DH), jnp.float32),
          jax.ShapeDtypeStruct((N, D), jnp.float32),
      ],
      name="tc_first",
  )(x, wl, wr, b)


def _mean2(aggA0, aggA1, aggB0, aggB1, cnt0, cnt1):
  cnt = jnp.maximum(cnt0 + cnt1, 1.0)
  meanA = (aggA0 + aggA1) / cnt
  meanB = (aggB0 + aggB1) / cnt
  return jnp.concatenate([meanA, meanB], axis=1)


def _tc_mid_body(aggA0_ref, aggA1_ref, aggB0_ref, aggB1_ref, cnt0_ref,
                 cnt1_ref, hr_ref, wl_ref, wr_ref, b_ref,
                 hla_ref, hlb_ref, hro_ref):
  mean = _mean2(aggA0_ref[0], aggA1_ref[0], aggB0_ref[0], aggB1_ref[0],
                cnt0_ref[0], cnt1_ref[0])
  h = jax.nn.relu(mean + hr_ref[...])
  hl = jnp.dot(h, wl_ref[...], preferred_element_type=jnp.float32)
  hla_ref[...] = hl[:, :DH]
  hlb_ref[...] = hl[:, DH:]
  hro_ref[...] = (
      jnp.dot(h, wr_ref[...], preferred_element_type=jnp.float32) + b_ref[...]
  )


def _tc_mid(aggA, aggB, cnt3, hr, wl, wr, b):
  grid = (N // BN,)
  return pl.pallas_call(
      _tc_mid_body,
      grid=grid,
      in_specs=[
          pl.BlockSpec((1, BN, DH), lambda i: (0, i, 0)),
          pl.BlockSpec((1, BN, DH), lambda i: (1, i, 0)),
          pl.BlockSpec((1, BN, DH), lambda i: (0, i, 0)),
          pl.BlockSpec((1, BN, DH), lambda i: (1, i, 0)),
          pl.BlockSpec((1, BN, 1), lambda i: (0, i, 0)),
          pl.BlockSpec((1, BN, 1), lambda i: (1, i, 0)),
          pl.BlockSpec((BN, D), lambda i: (i, 0)),
          pl.BlockSpec((D, D), lambda i: (0, 0)),
          pl.BlockSpec((D, D), lambda i: (0, 0)),
          pl.BlockSpec((1, D), lambda i: (0, 0)),
      ],
      out_specs=[
          pl.BlockSpec((BN, DH), lambda i: (i, 0)),
          pl.BlockSpec((BN, DH), lambda i: (i, 0)),
          pl.BlockSpec((BN, D), lambda i: (i, 0)),
      ],
      out_shape=[
          jax.ShapeDtypeStruct((N, DH), jnp.float32),
          jax.ShapeDtypeStruct((N, DH), jnp.float32),
          jax.ShapeDtypeStruct((N, D), jnp.float32),
      ],
      name="tc_mid",
  )(aggA, aggA, aggB, aggB, cnt3, cnt3, hr, wl, wr, b)


def _tc_last_body(aggA0_ref, aggA1_ref, aggB0_ref, aggB1_ref, cnt0_ref,
                  cnt1_ref, hr_ref, out_ref):
  mean = _mean2(aggA0_ref[0], aggA1_ref[0], aggB0_ref[0], aggB1_ref[0],
                cnt0_ref[0], cnt1_ref[0])
  out_ref[...] = mean + hr_ref[...]


def _tc_last(aggA, aggB, cnt3, hr):
  grid = (N // BN,)
  return pl.pallas_call(
      _tc_last_body,
      grid=grid,
      in_specs=[
          pl.BlockSpec((1, BN, DH), lambda i: (0, i, 0)),
          pl.BlockSpec((1, BN, DH), lambda i: (1, i, 0)),
          pl.BlockSpec((1, BN, DH), lambda i: (0, i, 0)),
          pl.BlockSpec((1, BN, DH), lambda i: (1, i, 0)),
          pl.BlockSpec((1, BN, 1), lambda i: (0, i, 0)),
          pl.BlockSpec((1, BN, 1), lambda i: (1, i, 0)),
          pl.BlockSpec((BN, D), lambda i: (i, 0)),
      ],
      out_specs=pl.BlockSpec((BN, D), lambda i: (i, 0)),
      out_shape=jax.ShapeDtypeStruct((N, D), jnp.float32),
      name="tc_last",
  )(aggA, aggA, aggB, aggB, cnt3, cnt3, hr)


@jax.jit
def kernel(x, edge_index, W1l, W1r, b1, W2l, W2r, b2, W3l, W3r, b3):
  pad = jnp.zeros(((PADROWS - NCHUNK) * K,), jnp.int32)
  src = jnp.concatenate([edge_index[0].astype(jnp.int32), pad]).reshape(
      PADROWS, K)
  dst = jnp.concatenate([edge_index[1].astype(jnp.int32), pad]).reshape(
      PADROWS, K)
  b1r = b1.reshape(1, D)
  b2r = b2.reshape(1, D)
  b3r = b3.reshape(1, D)

  hla, hlb, hr = _tc_first(x, W1l, W1r, b1r)
  aggA, aggB, cntw = _make_agg(True)(hla, hlb, src, dst)
  cnt3 = cntw.reshape(NC, NP, 1)

  hla, hlb, hr = _tc_mid(aggA, aggB, cnt3, hr, W2l, W2r, b2r)
  aggA, aggB = _make_agg(False)(hla, hlb, src, dst)

  hla, hlb, hr = _tc_mid(aggA, aggB, cnt3, hr, W3l, W3r, b3r)
  aggA, aggB = _make_agg(False)(hla, hlb, src, dst)

  return _tc_last(aggA, aggB, cnt3, hr)
